# baseline (speedup 1.0000x reference)
import functools

import jax
import jax.numpy as jnp
from jax import lax
from jax.experimental import pallas as pl
from jax.experimental.pallas import tpu as pltpu
from jax.experimental.pallas import tpu_sc as plsc

B, T, K, H = 1024, 10, 16, 256
NW = 32
GROUPS = B * T
GPW = GROUPS // NW
SPW = B // NW
GB = 8
RB = GB * K
NBATCH = GPW // GB
LANES = 16
C = H // LANES

_MESH = plsc.VectorSubcoreMesh(core_axis_name="c", subcore_axis_name="s")


def _sc_gather_sum(hist, s, r, ent, rel):
    @functools.partial(
        pl.kernel,
        mesh=_MESH,
        out_type=[
            jax.ShapeDtypeStruct((GROUPS, H), jnp.float32),
            jax.ShapeDtypeStruct((B, H), jnp.float32),
            jax.ShapeDtypeStruct((B, H), jnp.float32),
        ],
        scratch_types=[
            pltpu.VMEM((GPW * K,), jnp.int32),
            pltpu.VMEM((RB, H), jnp.float32),
            pltpu.VMEM((RB, H), jnp.float32),
            pltpu.VMEM((GB, H), jnp.float32),
            pltpu.VMEM((SPW,), jnp.int32),
            pltpu.VMEM((SPW, H), jnp.float32),
            pltpu.SemaphoreType.DMA,
            pltpu.SemaphoreType.DMA,
        ],
    )
    def k(hist_hbm, s_hbm, r_hbm, ent_hbm, rel_hbm,
          sums_hbm, se_hbm, re_hbm,
          idx_v, rows0, rows1, sums_v, sidx_v, srows_v, sem0, sem1):
        wid = lax.axis_index("s") * 2 + lax.axis_index("c")
        base_s = wid * SPW
        base_g = wid * GPW

        pltpu.sync_copy(hist_hbm.at[pl.ds(base_g * K, GPW * K)], idx_v)

        def gather_cp(bi, rows_ref, sem):
            return pltpu.make_async_copy(
                ent_hbm.at[idx_v.at[pl.ds(bi * RB, RB)]], rows_ref, sem)

        def finish_batch(bi, rows_ref, sem):
            gather_cp(bi, rows_ref, sem).wait()
            for g in range(GB):
                def row_body(rr, acc, g=g):
                    return tuple(
                        acc[c]
                        + rows_ref[g * K + 2 * rr, pl.ds(c * LANES, LANES)]
                        + rows_ref[g * K + 2 * rr + 1,
                                   pl.ds(c * LANES, LANES)]
                        for c in range(C)
                    )
                acc = tuple(
                    rows_ref[g * K, pl.ds(c * LANES, LANES)]
                    + rows_ref[g * K + 1, pl.ds(c * LANES, LANES)]
                    for c in range(C)
                )
                acc = lax.fori_loop(1, K // 2, row_body, acc)
                for c in range(C):
                    sums_v[g, pl.ds(c * LANES, LANES)] = acc[c]
            pltpu.sync_copy(sums_v, sums_hbm.at[pl.ds(base_g + bi * GB, GB)])

        gather_cp(0, rows0, sem0).start()

        def outer(j, carry):
            gather_cp(2 * j + 1, rows1, sem1).start()
            finish_batch(2 * j, rows0, sem0)

            @pl.when(j < NBATCH // 2 - 1)
            def _():
                gather_cp(2 * j + 2, rows0, sem0).start()

            finish_batch(2 * j + 1, rows1, sem1)
            return carry

        lax.fori_loop(0, NBATCH // 2, outer, 0)

        pltpu.sync_copy(s_hbm.at[pl.ds(base_s, SPW)], sidx_v)
        pltpu.async_copy(ent_hbm.at[sidx_v], srows_v, sem0).wait()
        pltpu.sync_copy(srows_v, se_hbm.at[pl.ds(base_s, SPW)])
        pltpu.sync_copy(r_hbm.at[pl.ds(base_s, SPW)], sidx_v)
        pltpu.async_copy(rel_hbm.at[sidx_v], srows_v, sem0).wait()
        pltpu.sync_copy(srows_v, re_hbm.at[pl.ds(base_s, SPW)])

    return k(hist, s, r, ent, rel)


def _tc_finish(sums2, s_e, r_e, W, b2):
    BB = 256

    def body(sums_ref, se_ref, re_ref, w_ref, b_ref, out_ref):
        x = sums_ref[...] * (1.0 / K)
        y = jnp.dot(x, w_ref[...], preferred_element_type=jnp.float32)
        y = jnp.maximum(y + b_ref[...], 0.0)
        out_ref[:, :, 0:H] = y.reshape(BB, T, H)
        out_ref[:, :, H:2 * H] = jnp.broadcast_to(
            se_ref[...][:, None, :], (BB, T, H))
        out_ref[:, :, 2 * H:3 * H] = jnp.broadcast_to(
            re_ref[...][:, None, :], (BB, T, H))

    return pl.pallas_call(
        body,
        grid=(B // BB,),
        in_specs=[
            pl.BlockSpec((BB * T, H), lambda i: (i, 0)),
            pl.BlockSpec((BB, H), lambda i: (i, 0)),
            pl.BlockSpec((BB, H), lambda i: (i, 0)),
            pl.BlockSpec((H, H), lambda i: (0, 0)),
            pl.BlockSpec((1, H), lambda i: (0, 0)),
        ],
        out_specs=pl.BlockSpec((BB, T, 3 * H), lambda i: (i, 0, 0)),
        out_shape=jax.ShapeDtypeStruct((B, T, 3 * H), jnp.float32),
    )(sums2, s_e, r_e, W, b2)


def kernel(s_hist, s, r, ent_embeds, rel_embeds, W, b):
    hist = s_hist.reshape(-1).astype(jnp.int32)
    sums, s_e, r_e = _sc_gather_sum(
        hist, s.astype(jnp.int32), r.astype(jnp.int32),
        ent_embeds, rel_embeds)
    return _tc_finish(sums, s_e, r_e, W, b.reshape(1, H))


# speedup vs baseline: 1.0529x; 1.0529x over previous
import functools

import jax
import jax.numpy as jnp
from jax import lax
from jax.experimental import pallas as pl
from jax.experimental.pallas import tpu as pltpu
from jax.experimental.pallas import tpu_sc as plsc

B, T, K, H = 1024, 10, 16, 256
NW = 32
GROUPS = B * T
GPW = GROUPS // NW
SPW = B // NW
GB = 8
RB = GB * K
NBATCH = GPW // GB
LANES = 16
C = H // LANES

_MESH = plsc.VectorSubcoreMesh(core_axis_name="c", subcore_axis_name="s")


def _sc_gather_sum(hist, s, r, ent, rel):
    @functools.partial(
        pl.kernel,
        mesh=_MESH,
        out_type=[
            jax.ShapeDtypeStruct((GROUPS, H), jnp.float32),
            jax.ShapeDtypeStruct((B, H), jnp.float32),
            jax.ShapeDtypeStruct((B, H), jnp.float32),
        ],
        scratch_types=[
            pltpu.VMEM((GPW * K,), jnp.int32),
            pltpu.VMEM((RB, H), jnp.float32),
            pltpu.VMEM((RB, H), jnp.float32),
            pltpu.VMEM((GB, H), jnp.float32),
            pltpu.VMEM((SPW,), jnp.int32),
            pltpu.VMEM((SPW, H), jnp.float32),
            pltpu.SemaphoreType.DMA,
            pltpu.SemaphoreType.DMA,
        ],
    )
    def k(hist_hbm, s_hbm, r_hbm, ent_hbm, rel_hbm,
          sums_hbm, se_hbm, re_hbm,
          idx_v, rows0, rows1, sums_v, sidx_v, srows_v, sem0, sem1):
        wid = lax.axis_index("s") * 2 + lax.axis_index("c")
        base_s = wid * SPW
        base_g = wid * GPW

        pltpu.sync_copy(hist_hbm.at[pl.ds(base_g * K, GPW * K)], idx_v)

        def gather_cp(bi, rows_ref, sem):
            return pltpu.make_async_copy(
                ent_hbm.at[idx_v.at[pl.ds(bi * RB, RB)]], rows_ref, sem)

        def finish_batch(bi, rows_ref, sem):
            gather_cp(bi, rows_ref, sem).wait()
            for g in range(GB):
                def row_body(rr, acc, g=g):
                    return tuple(
                        acc[c] + rows_ref[g * K + rr, pl.ds(c * LANES, LANES)]
                        for c in range(C)
                    )
                acc = tuple(
                    rows_ref[g * K, pl.ds(c * LANES, LANES)] for c in range(C)
                )
                acc = lax.fori_loop(1, K, row_body, acc)
                for c in range(C):
                    sums_v[g, pl.ds(c * LANES, LANES)] = acc[c]
            pltpu.sync_copy(sums_v, sums_hbm.at[pl.ds(base_g + bi * GB, GB)])

        gather_cp(0, rows0, sem0).start()

        def outer(j, carry):
            gather_cp(2 * j + 1, rows1, sem1).start()
            finish_batch(2 * j, rows0, sem0)

            @pl.when(j < NBATCH // 2 - 1)
            def _():
                gather_cp(2 * j + 2, rows0, sem0).start()

            finish_batch(2 * j + 1, rows1, sem1)
            return carry

        lax.fori_loop(0, NBATCH // 2, outer, 0)

        pltpu.sync_copy(s_hbm.at[pl.ds(base_s, SPW)], sidx_v)
        pltpu.async_copy(ent_hbm.at[sidx_v], srows_v, sem0).wait()
        pltpu.sync_copy(srows_v, se_hbm.at[pl.ds(base_s, SPW)])
        pltpu.sync_copy(r_hbm.at[pl.ds(base_s, SPW)], sidx_v)
        pltpu.async_copy(rel_hbm.at[sidx_v], srows_v, sem0).wait()
        pltpu.sync_copy(srows_v, re_hbm.at[pl.ds(base_s, SPW)])

    return k(hist, s, r, ent, rel)


def _tc_finish(sums2, s_e, r_e, W, b2):
    BB = 256

    def body(sums_ref, se_ref, re_ref, w_ref, b_ref, out_ref):
        x = sums_ref[...] * (1.0 / K)
        y = jnp.dot(x, w_ref[...], preferred_element_type=jnp.float32)
        y = jnp.maximum(y + b_ref[...], 0.0)
        out_ref[:, :, 0:H] = y.reshape(BB, T, H)
        out_ref[:, :, H:2 * H] = jnp.broadcast_to(
            se_ref[...][:, None, :], (BB, T, H))
        out_ref[:, :, 2 * H:3 * H] = jnp.broadcast_to(
            re_ref[...][:, None, :], (BB, T, H))

    return pl.pallas_call(
        body,
        grid=(B // BB,),
        in_specs=[
            pl.BlockSpec((BB * T, H), lambda i: (i, 0)),
            pl.BlockSpec((BB, H), lambda i: (i, 0)),
            pl.BlockSpec((BB, H), lambda i: (i, 0)),
            pl.BlockSpec((H, H), lambda i: (0, 0)),
            pl.BlockSpec((1, H), lambda i: (0, 0)),
        ],
        out_specs=pl.BlockSpec((BB, T, 3 * H), lambda i: (i, 0, 0)),
        out_shape=jax.ShapeDtypeStruct((B, T, 3 * H), jnp.float32),
    )(sums2, s_e, r_e, W, b2)


def kernel(s_hist, s, r, ent_embeds, rel_embeds, W, b):
    hist = s_hist.reshape(-1).astype(jnp.int32)
    sums, s_e, r_e = _sc_gather_sum(
        hist, s.astype(jnp.int32), r.astype(jnp.int32),
        ent_embeds, rel_embeds)
    return _tc_finish(sums, s_e, r_e, W, b.reshape(1, H))
